# Initial kernel scaffold; baseline (speedup 1.0000x reference)
import functools

import jax
import jax.numpy as jnp
from jax import lax
from jax.experimental import pallas as pl
from jax.experimental.pallas import tpu as pltpu
from jax.experimental.pallas import tpu_sc as plsc

N_NODES = 10000
D_IN = 128
HIDDEN = 100
HP = 112
D_OUT = 16
N_EDGES = 320000

NC, NS = 2, 16
NW = NC * NS
E_PER_TILE = N_EDGES // NW
CHUNK = 125
N_CHUNKS = E_PER_TILE // CHUNK
ROWS_PER_TILE = N_NODES // NS


def _tc_pre(x, w_pad):
    def body(x_ref, w_ref, o_ref):
        o_ref[...] = lax.dot_general(
            x_ref[...], w_ref[...], (((1,), (0,)), ((), ())),
            precision=lax.Precision.HIGHEST,
            preferred_element_type=jnp.float32)

    return pl.pallas_call(
        body,
        out_shape=jax.ShapeDtypeStruct((N_NODES, HP), jnp.float32),
    )(x, w_pad)


def _sc_aggregate(y, src3, dst3):
    mesh = plsc.VectorSubcoreMesh(core_axis_name="c", subcore_axis_name="s")

    @functools.partial(
        pl.kernel,
        out_type=jax.ShapeDtypeStruct((NC, N_NODES, HP), jnp.float32),
        mesh=mesh,
        scratch_types=[
            pltpu.VMEM((N_CHUNKS, CHUNK), jnp.int32),
            pltpu.VMEM((N_CHUNKS, CHUNK), jnp.int32),
            pltpu.VMEM((CHUNK, HP), jnp.float32),
            pltpu.VMEM_SHARED((N_NODES, HP), jnp.float32),
        ],
    )
    def k(y_hbm, src_hbm, dst_hbm, out_hbm, src_v, dst_v, rows_v, acc_sh):
        c = lax.axis_index("c")
        s = lax.axis_index("s")
        w = c * NS + s

        pltpu.sync_copy(src_hbm.at[w], src_v)
        pltpu.sync_copy(dst_hbm.at[w], dst_v)

        @pl.loop(0, CHUNK)
        def _(i):
            @pl.loop(0, HP, step=16)
            def _(j):
                rows_v[i, pl.ds(j, 16)] = jnp.zeros((16,), jnp.float32)

        for z in range(ROWS_PER_TILE // CHUNK):
            pltpu.sync_copy(
                rows_v,
                acc_sh.at[pl.ds(s * ROWS_PER_TILE + z * CHUNK, CHUNK)])
        plsc.subcore_barrier()

        @pl.loop(0, N_CHUNKS)
        def _(i):
            pltpu.sync_copy(y_hbm.at[src_v.at[i]], rows_v)
            pltpu.sync_copy(rows_v, acc_sh.at[dst_v.at[i]], add=True)

        plsc.subcore_barrier()
        pltpu.sync_copy(
            acc_sh.at[pl.ds(s * ROWS_PER_TILE, ROWS_PER_TILE)],
            out_hbm.at[c, pl.ds(s * ROWS_PER_TILE, ROWS_PER_TILE)])

    return k(y, src3, dst3)


def _tc_head(parts, b_pad, w1t_pad, b1_pad, gamma_pad, beta_pad, w2t_pad, b2):
    def body(p_ref, b_ref, w1_ref, b1_ref, g_ref, be_ref, w2_ref, b2_ref,
             o_ref):
        z = p_ref[0] + p_ref[1] + b_ref[...]
        h = jnp.maximum(z, 0.0)
        h = lax.dot_general(
            h, w1_ref[...], (((1,), (0,)), ((), ())),
            precision=lax.Precision.HIGHEST,
            preferred_element_type=jnp.float32) + b1_ref[...]
        h = jnp.maximum(h, 0.0)
        mean = jnp.mean(h, axis=0, keepdims=True)
        var = jnp.mean((h - mean) ** 2, axis=0, keepdims=True)
        hn = (h - mean) * lax.rsqrt(var + 1e-5) * g_ref[...] + be_ref[...]
        logits = lax.dot_general(
            hn, w2_ref[...], (((1,), (0,)), ((), ())),
            precision=lax.Precision.HIGHEST,
            preferred_element_type=jnp.float32) + b2_ref[...]
        m = jnp.max(logits, axis=1, keepdims=True)
        e = jnp.exp(logits - m)
        o_ref[...] = e / jnp.sum(e, axis=1, keepdims=True)

    return pl.pallas_call(
        body,
        out_shape=jax.ShapeDtypeStruct((N_NODES, D_OUT), jnp.float32),
    )(parts, b_pad, w1t_pad, b1_pad, gamma_pad, beta_pad, w2t_pad, b2)


def kernel(x, edge_index, W_mpnn, b_mpnn, W1, b1, gamma, beta, W2, b2):
    src = edge_index[0].astype(jnp.int32).reshape(NW, N_CHUNKS, CHUNK)
    dst = edge_index[1].astype(jnp.int32).reshape(NW, N_CHUNKS, CHUNK)

    w_pad = jnp.zeros((D_IN, HP), jnp.float32).at[:, :HIDDEN].set(W_mpnn.T)
    b_pad = jnp.zeros((1, HP), jnp.float32).at[0, :HIDDEN].set(b_mpnn)
    w1t_pad = jnp.zeros((HP, HP), jnp.float32).at[:HIDDEN, :HIDDEN].set(W1.T)
    b1_pad = jnp.zeros((1, HP), jnp.float32).at[0, :HIDDEN].set(b1)
    gamma_pad = jnp.zeros((1, HP), jnp.float32).at[0, :HIDDEN].set(gamma)
    beta_pad = jnp.zeros((1, HP), jnp.float32).at[0, :HIDDEN].set(beta)
    w2t_pad = jnp.zeros((HP, D_OUT), jnp.float32).at[:HIDDEN, :].set(W2.T)
    b2_row = b2.reshape(1, D_OUT)

    y = _tc_pre(x, w_pad)
    parts = _sc_aggregate(y, src, dst)
    return _tc_head(parts, b_pad, w1t_pad, b1_pad, gamma_pad, beta_pad,
                    w2t_pad, b2_row)


# speedup vs baseline: 8.0014x; 8.0014x over previous
import functools

import jax
import jax.numpy as jnp
from jax import lax
from jax.experimental import pallas as pl
from jax.experimental.pallas import tpu as pltpu
from jax.experimental.pallas import tpu_sc as plsc

N_NODES = 10000
NP = 10240
D_IN = 128
HIDDEN = 100
HP = 128
D_OUT = 16
N_EDGES = 320000

NC, NS = 2, 16
NW = NC * NS
E_PER_TILE = N_EDGES // NW
CHUNK = 125
N_CHUNKS = E_PER_TILE // CHUNK
ZROWS = 64
ROWS_PER_TILE = NP // NS


def _sc_aggregate(x, src3, dst3):
    mesh = plsc.VectorSubcoreMesh(core_axis_name="c", subcore_axis_name="s")

    @functools.partial(
        pl.kernel,
        out_type=jax.ShapeDtypeStruct((NC, NP, D_IN), jnp.float32),
        mesh=mesh,
        scratch_types=[
            pltpu.VMEM((N_CHUNKS, CHUNK), jnp.int32),
            pltpu.VMEM((N_CHUNKS, CHUNK), jnp.int32),
            pltpu.VMEM((CHUNK, D_IN), jnp.float32),
            pltpu.VMEM_SHARED((NP, D_IN), jnp.float32),
        ],
    )
    def k(x_hbm, src_hbm, dst_hbm, out_hbm, src_v, dst_v, rows_v, acc_sh):
        c = lax.axis_index("c")
        s = lax.axis_index("s")
        w = c * NS + s

        pltpu.sync_copy(src_hbm.at[w], src_v)
        pltpu.sync_copy(dst_hbm.at[w], dst_v)

        @pl.loop(0, ZROWS)
        def _(i):
            @pl.loop(0, D_IN, step=16)
            def _(j):
                rows_v[i, pl.ds(j, 16)] = jnp.zeros((16,), jnp.float32)

        for z in range(ROWS_PER_TILE // ZROWS):
            pltpu.sync_copy(
                rows_v.at[pl.ds(0, ZROWS)],
                acc_sh.at[pl.ds(s * ROWS_PER_TILE + z * ZROWS, ZROWS)])
        plsc.subcore_barrier()

        @pl.loop(0, N_CHUNKS)
        def _(i):
            pltpu.sync_copy(x_hbm.at[src_v.at[i]], rows_v)
            pltpu.sync_copy(rows_v, acc_sh.at[dst_v.at[i]], add=True)

        plsc.subcore_barrier()
        pltpu.sync_copy(
            acc_sh.at[pl.ds(s * ROWS_PER_TILE, ROWS_PER_TILE)],
            out_hbm.at[c, pl.ds(s * ROWS_PER_TILE, ROWS_PER_TILE)])

    return k(x, src3, dst3)


def _tc_head(parts, wmt_pad, b_pad, w1t_pad, b1_pad, gamma_pad, beta_pad,
             w2t_pad, b2):
    def body(p_ref, wm_ref, b_ref, w1_ref, b1_ref, g_ref, be_ref, w2_ref,
             b2_ref, o_ref):
        agg = p_ref[0, :N_NODES] + p_ref[1, :N_NODES]
        h = lax.dot_general(
            agg, wm_ref[...], (((1,), (0,)), ((), ())),
            precision=lax.Precision.HIGHEST,
            preferred_element_type=jnp.float32) + b_ref[...]
        h = jnp.maximum(h, 0.0)
        h = lax.dot_general(
            h, w1_ref[...], (((1,), (0,)), ((), ())),
            precision=lax.Precision.HIGHEST,
            preferred_element_type=jnp.float32) + b1_ref[...]
        h = jnp.maximum(h, 0.0)
        mean = jnp.mean(h, axis=0, keepdims=True)
        var = jnp.mean((h - mean) ** 2, axis=0, keepdims=True)
        hn = (h - mean) * lax.rsqrt(var + 1e-5) * g_ref[...] + be_ref[...]
        logits = lax.dot_general(
            hn, w2_ref[...], (((1,), (0,)), ((), ())),
            precision=lax.Precision.HIGHEST,
            preferred_element_type=jnp.float32) + b2_ref[...]
        m = jnp.max(logits, axis=1, keepdims=True)
        e = jnp.exp(logits - m)
        o_ref[...] = e / jnp.sum(e, axis=1, keepdims=True)

    return pl.pallas_call(
        body,
        out_shape=jax.ShapeDtypeStruct((N_NODES, D_OUT), jnp.float32),
    )(parts, wmt_pad, b_pad, w1t_pad, b1_pad, gamma_pad, beta_pad, w2t_pad,
      b2)


def kernel(x, edge_index, W_mpnn, b_mpnn, W1, b1, gamma, beta, W2, b2):
    src = edge_index[0].astype(jnp.int32).reshape(NW, N_CHUNKS, CHUNK)
    dst = edge_index[1].astype(jnp.int32).reshape(NW, N_CHUNKS, CHUNK)

    wmt_pad = jnp.zeros((D_IN, HP), jnp.float32).at[:, :HIDDEN].set(W_mpnn.T)
    b_pad = jnp.zeros((1, HP), jnp.float32).at[0, :HIDDEN].set(b_mpnn)
    w1t_pad = jnp.zeros((HP, HP), jnp.float32).at[:HIDDEN, :HIDDEN].set(W1.T)
    b1_pad = jnp.zeros((1, HP), jnp.float32).at[0, :HIDDEN].set(b1)
    gamma_pad = jnp.zeros((1, HP), jnp.float32).at[0, :HIDDEN].set(gamma)
    beta_pad = jnp.zeros((1, HP), jnp.float32).at[0, :HIDDEN].set(beta)
    w2t_pad = jnp.zeros((HP, D_OUT), jnp.float32).at[:HIDDEN, :].set(W2.T)
    b2_row = b2.reshape(1, D_OUT)

    parts = _sc_aggregate(x, src, dst)
    return _tc_head(parts, wmt_pad, b_pad, w1t_pad, b1_pad, gamma_pad,
                    beta_pad, w2t_pad, b2_row)
